# Initial kernel scaffold; baseline (speedup 1.0000x reference)
import jax
import jax.numpy as jnp
from jax.experimental import pallas as pl


def kernel(x_subject, x_electrode, edge_e2s, edge_s2s, W_proj_s, b_proj_s, W_proj_e, b_proj_e, att_src_e2s, att_dst_e2s, att_src_s2s, att_dst_s2s, k_lin_w, k_lin_b, q_sem, W_out, b_out):
    raise NotImplementedError("write your pallas kernel here")


# speedup vs baseline: 22.3800x; 22.3800x over previous
import functools

import jax
import jax.numpy as jnp
from jax import lax
from jax.experimental import pallas as pl
from jax.experimental.pallas import tpu as pltpu
from jax.experimental.pallas import tpu_sc as plsc

N = 10000
D = 128
NH = 8
DH = 16
E = 320000
NC, NS = 2, 16
EPT = E // NS
B = 32
NB = EPT // B
BN = 400
NBLK = N // BN
SPAN = 640
ZR = 16


def _proj_body(x_ref, w_ref, b_ref, msrc_ref, mdst_ref,
               hp_ref, adst_ref):
    z112 = jnp.zeros((BN, 112), jnp.float32)
    for t in range(2):
        h = jnp.dot(x_ref[t], w_ref[t],
                    preferred_element_type=jnp.float32) + b_ref[t]
        asrc = jnp.dot(h, msrc_ref[t], preferred_element_type=jnp.float32)
        hp_ref[t] = jnp.concatenate([h, asrc, z112], axis=1)
        if t == 1:
            for r in range(2):
                ad = jnp.dot(h, mdst_ref[r], preferred_element_type=jnp.float32)
                adst_ref[r] = jnp.concatenate([ad, z112], axis=1)


def _run_proj(x_stack, w_stack, b_stack, msrc_stack, mdst_stack):
    return pl.pallas_call(
        _proj_body,
        grid=(NBLK,),
        in_specs=[
            pl.BlockSpec((2, BN, D), lambda i: (0, i, 0)),
            pl.BlockSpec((2, D, D), lambda i: (0, 0, 0)),
            pl.BlockSpec((2, 1, D), lambda i: (0, 0, 0)),
            pl.BlockSpec((2, D, 16), lambda i: (0, 0, 0)),
            pl.BlockSpec((2, D, 16), lambda i: (0, 0, 0)),
        ],
        out_specs=[
            pl.BlockSpec((2, BN, 256), lambda i: (0, i, 0)),
            pl.BlockSpec((2, BN, 128), lambda i: (0, i, 0)),
        ],
        out_shape=[
            jax.ShapeDtypeStruct((2, N, 256), jnp.float32),
            jax.ShapeDtypeStruct((2, N, 128), jnp.float32),
        ],
    )(x_stack, w_stack, b_stack, msrc_stack, mdst_stack)


def _lane_bcast(v, lane):
    idx = jnp.full((16, 1), lane, jnp.int32)
    return lax.gather(
        v, idx,
        dimension_numbers=lax.GatherDimensionNumbers(
            offset_dims=(), collapsed_slice_dims=(0,), start_index_map=(0,)),
        slice_sizes=(1,),
        mode=lax.GatherScatterMode.PROMISE_IN_BOUNDS)


def _edge_body(src_hbm, dst_hbm, hp_hbm, adst_hbm,
               num_hbm, den_hbm, ex_hbm,
               src_v, dsta_v, dst_v, hp_rows, adst_v, out_buf, exbuf,
               acc, sem):
    c = lax.axis_index("c")
    s = lax.axis_index("s")
    coff = c * N
    row0 = s * SPAN
    nchunk = jnp.where(s == NS - 1, (N - 15 * SPAN) // ZR, SPAN // ZR)

    zv = jnp.zeros((16,), jnp.float32)

    def zrow(j, _):
        for k in range(D // 16):
            out_buf[j, pl.ds(16 * k, 16)] = zv
        return _
    lax.fori_loop(0, B, zrow, None)

    def zero_span(k, _):
        pltpu.sync_copy(out_buf.at[pl.ds(0, ZR)],
                        acc.at[pl.ds(row0 + k * ZR, ZR)])
        return _
    lax.fori_loop(0, nchunk, zero_span, None)
    plsc.subcore_barrier()

    def batch_num(i, _):
        base = c * E + s * EPT + i * B
        pltpu.sync_copy(src_hbm.at[pl.ds(base, B)], src_v)
        pltpu.sync_copy(dst_hbm.at[pl.ds(base, B)], dst_v)
        for k in range(B // 16):
            sl = pl.ds(16 * k, 16)
            src_v[sl] = src_v[sl] + coff
            dsta_v[sl] = dst_v[sl] + coff
        cp0 = pltpu.async_copy(hp_hbm.at[src_v], hp_rows, sem)
        cp1 = pltpu.async_copy(adst_hbm.at[dsta_v], adst_v, sem)
        cp0.wait()
        cp1.wait()

        def edge(e, _):
            a = hp_rows[e, pl.ds(D, 16)] + adst_v[e, pl.ds(0, 16)]
            a = jnp.maximum(a, 0.0) + 0.2 * jnp.minimum(a, 0.0)
            ex = jnp.exp(a)
            exbuf[e, :] = ex
            for hh in range(NH):
                w = _lane_bcast(ex, hh)
                out_buf[e, pl.ds(16 * hh, 16)] = hp_rows[e, pl.ds(16 * hh, 16)] * w
            return _
        lax.fori_loop(0, B, edge, None)
        pltpu.sync_copy(out_buf, acc.at[dst_v], add=True)
        pltpu.sync_copy(exbuf, ex_hbm.at[pl.ds(base, B)])
        return _
    lax.fori_loop(0, NB, batch_num, None)
    plsc.subcore_barrier()

    lax.fori_loop(0, B, zrow, None)

    def flush_num(k, _):
        rows = pl.ds(row0 + k * ZR, ZR)
        pltpu.sync_copy(acc.at[rows], num_hbm.at[c, rows])
        pltpu.sync_copy(out_buf.at[pl.ds(0, ZR)], acc.at[rows])
        return _
    lax.fori_loop(0, nchunk, flush_num, None)
    plsc.subcore_barrier()


    def batch_den(i, _):
        base = c * E + s * EPT + i * B
        pltpu.sync_copy(dst_hbm.at[pl.ds(base, B)], dst_v)
        pltpu.sync_copy(ex_hbm.at[pl.ds(base, B)], exbuf)

        def edge(e, _):
            out_buf[e, pl.ds(0, 16)] = exbuf[e, :]
            return _
        lax.fori_loop(0, B, edge, None)
        pltpu.sync_copy(out_buf, acc.at[dst_v], add=True)
        return _
    lax.fori_loop(0, NB, batch_den, None)
    plsc.subcore_barrier()

    def flush_den(k, _):
        rows = pl.ds(row0 + k * ZR, ZR)
        pltpu.sync_copy(acc.at[rows], den_hbm.at[c, rows])
        return _
    lax.fori_loop(0, nchunk, flush_den, None)


def _run_edges(src2, dst2, hp2, adst2):
    mesh = plsc.VectorSubcoreMesh(core_axis_name="c", subcore_axis_name="s")
    f = functools.partial(
        pl.kernel,
        out_type=[
            jax.ShapeDtypeStruct((2, N, 128), jnp.float32),
            jax.ShapeDtypeStruct((2, N, 128), jnp.float32),
            jax.ShapeDtypeStruct((2 * E, 16), jnp.float32),
        ],
        mesh=mesh,
        scratch_types=[
            pltpu.VMEM((B,), jnp.int32),
            pltpu.VMEM((B,), jnp.int32),
            pltpu.VMEM((B,), jnp.int32),
            pltpu.VMEM((B, 2 * D), jnp.float32),
            pltpu.VMEM((B, D), jnp.float32),
            pltpu.VMEM((B, D), jnp.float32),
            pltpu.VMEM((B, 16), jnp.float32),
            pltpu.VMEM_SHARED((N, D), jnp.float32),
            pltpu.SemaphoreType.DMA,
        ],
    )(_edge_body)
    return f(src2, dst2, hp2, adst2)


def _epi_a_body(num_ref, den_ref, r16_ref, klin_ref, klinb_ref,
                o_ref, p_ref):
    for r in range(2):
        den128 = jnp.dot(den_ref[r][:, :16], r16_ref[...],
                         preferred_element_type=jnp.float32)
        o = jnp.maximum(num_ref[r] / (den128 + 1e-16), 0.0)
        o_ref[r] = o
        kmat = jnp.tanh(jnp.dot(o, klin_ref[...],
                                preferred_element_type=jnp.float32)
                        + klinb_ref[...])
        p_ref[0, r] = jnp.sum(kmat, axis=0)


def _run_epi_a(num, den, r16, klin, klinb):
    return pl.pallas_call(
        _epi_a_body,
        grid=(NBLK,),
        in_specs=[
            pl.BlockSpec((2, BN, 128), lambda i: (0, i, 0)),
            pl.BlockSpec((2, BN, 128), lambda i: (0, i, 0)),
            pl.BlockSpec((16, 128), lambda i: (0, 0)),
            pl.BlockSpec((D, D), lambda i: (0, 0)),
            pl.BlockSpec((1, D), lambda i: (0, 0)),
        ],
        out_specs=[
            pl.BlockSpec((2, BN, 128), lambda i: (0, i, 0)),
            pl.BlockSpec((1, 2, 128), lambda i: (i, 0, 0)),
        ],
        out_shape=[
            jax.ShapeDtypeStruct((2, N, 128), jnp.float32),
            jax.ShapeDtypeStruct((NBLK, 2, 128), jnp.float32),
        ],
    )(num, den, r16, klin, klinb)


def _epi_b_body(p_ref, q_ref, o_ref, wp_ref, bp_ref, out_ref, attn_s):
    i = pl.program_id(0)

    @pl.when(i == 0)
    def _():
        psum = jnp.sum(p_ref[...], axis=0)
        s2 = jnp.sum(psum * q_ref[...], axis=1, keepdims=True) / N
        m = jnp.max(s2, axis=0, keepdims=True)
        e = jnp.exp(s2 - m)
        attn = e / jnp.sum(e, axis=0, keepdims=True)
        attn_s[...] = jnp.broadcast_to(attn, (2, 128))

    osub = o_ref[0] * attn_s[0:1, :] + o_ref[1] * attn_s[1:2, :]
    out_ref[...] = jnp.dot(osub, wp_ref[...],
                           preferred_element_type=jnp.float32) + bp_ref[...]


def _run_epi_b(p, q2, o, wp, bp):
    return pl.pallas_call(
        _epi_b_body,
        grid=(NBLK,),
        in_specs=[
            pl.BlockSpec((NBLK, 2, 128), lambda i: (0, 0, 0)),
            pl.BlockSpec((1, 128), lambda i: (0, 0)),
            pl.BlockSpec((2, BN, 128), lambda i: (0, i, 0)),
            pl.BlockSpec((D, D), lambda i: (0, 0)),
            pl.BlockSpec((1, D), lambda i: (0, 0)),
        ],
        out_specs=pl.BlockSpec((BN, 128), lambda i: (i, 0)),
        out_shape=jax.ShapeDtypeStruct((N, 128), jnp.float32),
        scratch_shapes=[pltpu.VMEM((2, 128), jnp.float32)],
    )(p, q2, o, wp, bp)


def _att_mat(att):
    eye = jnp.eye(NH, 16, dtype=att.dtype)
    return (att[:, :, None] * eye[:, None, :]).reshape(D, 16)


def kernel(x_subject, x_electrode, edge_e2s, edge_s2s, W_proj_s, b_proj_s,
           W_proj_e, b_proj_e, att_src_e2s, att_dst_e2s, att_src_s2s,
           att_dst_s2s, k_lin_w, k_lin_b, q_sem, W_out, b_out):
    x_stack = jnp.stack([x_electrode, x_subject])
    w_stack = jnp.stack([W_proj_e, W_proj_s])
    b_stack = jnp.stack([b_proj_e, b_proj_s]).reshape(2, 1, D)
    msrc_stack = jnp.stack([_att_mat(att_src_e2s), _att_mat(att_src_s2s)])
    mdst_stack = jnp.stack([_att_mat(att_dst_e2s), _att_mat(att_dst_s2s)])

    hp, adst = _run_proj(x_stack, w_stack, b_stack, msrc_stack, mdst_stack)

    src2 = jnp.concatenate([edge_e2s[0], edge_s2s[0]])
    dst2 = jnp.concatenate([edge_e2s[1], edge_s2s[1]])
    num, den, _unused_ex = _run_edges(src2, dst2,
                                      hp.reshape(2 * N, 256),
                                      adst.reshape(2 * N, 128))

    r16 = jnp.repeat(jnp.eye(NH, dtype=jnp.float32), DH, axis=1)
    r16 = jnp.concatenate([r16, jnp.zeros((8, 128), jnp.float32)], axis=0)
    o, p = _run_epi_a(num, den, r16, k_lin_w, k_lin_b.reshape(1, D))

    wp = jnp.zeros((D, D), jnp.float32).at[:, :2].set(W_out)
    bp = jnp.zeros((1, D), jnp.float32).at[0, :2].set(b_out)
    out_pad = _run_epi_b(p, q_sem.reshape(1, D), o, wp, bp)
    return out_pad[:, :2]


# speedup vs baseline: 75.6472x; 3.3801x over previous
import functools

import jax
import jax.numpy as jnp
from jax import lax
from jax.experimental import pallas as pl
from jax.experimental.pallas import tpu as pltpu
from jax.experimental.pallas import tpu_sc as plsc

N = 10000
D = 128
NH = 8
DH = 16
E = 320000
NC, NS = 2, 16
EPT = E // NS
B = 32
NB = EPT // B
BN = 400
NBLK = N // BN
SPAN = 640
ZR = 16


def _proj_body(x_ref, w_ref, b_ref, msrc_ref, mdst_ref,
               hp_ref, adst_ref):
    z112 = jnp.zeros((BN, 112), jnp.float32)
    for t in range(2):
        h = jnp.dot(x_ref[t], w_ref[t],
                    preferred_element_type=jnp.float32) + b_ref[t]
        asrc = jnp.dot(h, msrc_ref[t], preferred_element_type=jnp.float32)
        hp_ref[t] = jnp.concatenate([h, asrc, z112], axis=1)
        if t == 1:
            for r in range(2):
                ad = jnp.dot(h, mdst_ref[r], preferred_element_type=jnp.float32)
                adst_ref[r] = jnp.concatenate([ad, z112], axis=1)


def _run_proj(x_stack, w_stack, b_stack, msrc_stack, mdst_stack):
    return pl.pallas_call(
        _proj_body,
        grid=(NBLK,),
        in_specs=[
            pl.BlockSpec((2, BN, D), lambda i: (0, i, 0)),
            pl.BlockSpec((2, D, D), lambda i: (0, 0, 0)),
            pl.BlockSpec((2, 1, D), lambda i: (0, 0, 0)),
            pl.BlockSpec((2, D, 16), lambda i: (0, 0, 0)),
            pl.BlockSpec((2, D, 16), lambda i: (0, 0, 0)),
        ],
        out_specs=[
            pl.BlockSpec((2, BN, 256), lambda i: (0, i, 0)),
            pl.BlockSpec((2, BN, 128), lambda i: (0, i, 0)),
        ],
        out_shape=[
            jax.ShapeDtypeStruct((2, N, 256), jnp.float32),
            jax.ShapeDtypeStruct((2, N, 128), jnp.float32),
        ],
    )(x_stack, w_stack, b_stack, msrc_stack, mdst_stack)


def _lane_bcast(v, lane):
    idx = jnp.full((16, 1), lane, jnp.int32)
    return lax.gather(
        v, idx,
        dimension_numbers=lax.GatherDimensionNumbers(
            offset_dims=(), collapsed_slice_dims=(0,), start_index_map=(0,)),
        slice_sizes=(1,),
        mode=lax.GatherScatterMode.PROMISE_IN_BOUNDS)


def _edge_body(src_hbm, dst_hbm, hp_hbm, adst_hbm,
               num_hbm, den_hbm, ex_hbm,
               s0, s1, d0, d1, d2, d3, da0, da1,
               hp0, hp1, ad0, ad1, ob0, ob1, ex0, ex1,
               acc,
               isem0, isem1, isem2, isem3, gsem0, gsem1,
               ssem0, ssem1, xsem0, xsem1):
    c = lax.axis_index("c")
    s = lax.axis_index("s")
    coff = c * N
    row0 = s * SPAN
    nchunk = jnp.where(s == NS - 1, (N - 15 * SPAN) // ZR, SPAN // ZR)
    ebase = c * E + s * EPT

    srcv = (s0, s1)
    dstv = (d0, d1, d2, d3)
    dstav = (da0, da1)
    hpv = (hp0, hp1)
    adv = (ad0, ad1)
    obv = (ob0, ob1)
    exv = (ex0, ex1)
    isem = (isem0, isem1, isem2, isem3)
    gsem = (gsem0, gsem1)
    ssem = (ssem0, ssem1)
    xsem = (xsem0, xsem1)

    zv = jnp.zeros((16,), jnp.float32)

    def zero_obufs():
        def zrow(j, _):
            for k in range(D // 16):
                ob0[j, pl.ds(16 * k, 16)] = zv
                ob1[j, pl.ds(16 * k, 16)] = zv
            return _
        lax.fori_loop(0, B, zrow, None)

    zero_obufs()

    def zero_span(k, _):
        pltpu.sync_copy(ob0.at[pl.ds(0, ZR)],
                        acc.at[pl.ds(row0 + k * ZR, ZR)])
        return _
    lax.fori_loop(0, nchunk, zero_span, None)
    plsc.subcore_barrier()

    def a_issue_idx(i, q, b):
        base = ebase + i * B
        pltpu.async_copy(src_hbm.at[pl.ds(base, B)], srcv[b], isem[q])
        pltpu.async_copy(dst_hbm.at[pl.ds(base, B)], dstv[q], isem[q])

    def a_wait_idx_start_gather(q, b):
        pltpu.make_async_copy(src_hbm.at[pl.ds(0, B)], srcv[b], isem[q]).wait()
        pltpu.make_async_copy(dst_hbm.at[pl.ds(0, B)], dstv[q], isem[q]).wait()
        for k in range(B // 16):
            sl = pl.ds(16 * k, 16)
            dstav[b][sl] = dstv[q][sl] + coff
        pltpu.async_copy(hp_hbm.at[srcv[b]], hpv[b], gsem[b])
        pltpu.async_copy(adst_hbm.at[dstav[b]], adv[b], gsem[b])

    def a_wait_gather(b):
        pltpu.make_async_copy(hp_hbm.at[srcv[b]], hpv[b], gsem[b]).wait()
        pltpu.make_async_copy(adst_hbm.at[dstav[b]], adv[b], gsem[b]).wait()

    def a_wait_store(b):
        pltpu.make_async_copy(obv[b], acc.at[dstv[b]], ssem[b]).wait()
        pltpu.make_async_copy(ex_hbm.at[pl.ds(0, B)], exv[b], xsem[b]).wait()

    def a_compute(i, q, b):
        base = ebase + i * B
        for e in range(B):
            av = hpv[b][e, pl.ds(D, 16)] + adv[b][e, pl.ds(0, 16)]
            av = jnp.maximum(av, 0.0) + 0.2 * jnp.minimum(av, 0.0)
            exl = jnp.exp(av)
            exv[b][e, :] = exl
            for hh in range(NH):
                w = _lane_bcast(exl, hh)
                obv[b][e, pl.ds(16 * hh, 16)] = (
                    hpv[b][e, pl.ds(16 * hh, 16)] * w)
        pltpu.async_copy(obv[b], acc.at[dstv[q]], ssem[b], add=True)
        pltpu.async_copy(exv[b], ex_hbm.at[pl.ds(base, B)], xsem[b])

    a_issue_idx(0, 0, 0)
    a_wait_idx_start_gather(0, 0)
    a_issue_idx(1, 1, 1)
    a_wait_idx_start_gather(1, 1)
    a_wait_gather(0)
    a_issue_idx(2, 2, 0)
    a_compute(0, 0, 0)
    a_wait_idx_start_gather(2, 0)
    a_wait_gather(1)
    a_issue_idx(3, 3, 1)
    a_compute(1, 1, 1)
    a_wait_idx_start_gather(3, 1)

    def a_quad(g, _):
        i0 = 2 + 4 * g
        for k in range(4):
            q = (2 + k) % 4
            b = k % 2
            a_wait_gather(b)
            a_wait_store(b)
            a_issue_idx(i0 + k + 2, (q + 2) % 4, b)
            a_compute(i0 + k, q, b)
            a_wait_idx_start_gather((q + 2) % 4, b)
        return _
    lax.fori_loop(0, (NB - 5) // 4, a_quad, None)

    a_wait_gather(0)
    a_wait_store(0)
    a_issue_idx(NB - 1, 0, 0)
    a_compute(NB - 3, 2, 0)
    a_wait_idx_start_gather(0, 0)
    a_wait_gather(1)
    a_wait_store(1)
    a_compute(NB - 2, 3, 1)
    a_wait_gather(0)
    a_wait_store(0)
    a_compute(NB - 1, 0, 0)
    a_wait_store(0)
    a_wait_store(1)
    plsc.subcore_barrier()

    zero_obufs()

    def flush_num(k, _):
        rows = pl.ds(row0 + k * ZR, ZR)
        pltpu.sync_copy(acc.at[rows], num_hbm.at[c, rows])
        pltpu.sync_copy(ob0.at[pl.ds(0, ZR)], acc.at[rows])
        return _
    lax.fori_loop(0, nchunk, flush_num, None)
    plsc.subcore_barrier()

    def b_issue(i, q, b):
        base = ebase + i * B
        pltpu.async_copy(dst_hbm.at[pl.ds(base, B)], dstv[q], isem[q])
        pltpu.async_copy(ex_hbm.at[pl.ds(base, B)], exv[b], gsem[b])

    def b_wait_loads(q, b):
        pltpu.make_async_copy(dst_hbm.at[pl.ds(0, B)], dstv[q], isem[q]).wait()
        pltpu.make_async_copy(ex_hbm.at[pl.ds(0, B)], exv[b], gsem[b]).wait()

    def b_wait_store(b):
        pltpu.make_async_copy(obv[b], acc.at[dstv[b]], ssem[b]).wait()

    def b_compute(q, b):
        for e in range(B):
            obv[b][e, pl.ds(0, 16)] = exv[b][e, :]
        pltpu.async_copy(obv[b], acc.at[dstv[q]], ssem[b], add=True)

    b_issue(0, 0, 0)
    b_issue(1, 1, 1)
    b_wait_loads(0, 0)
    b_compute(0, 0)
    b_issue(2, 2, 0)
    b_wait_loads(1, 1)
    b_compute(1, 1)
    b_issue(3, 3, 1)

    def b_quad(g, _):
        i0 = 2 + 4 * g
        for k in range(4):
            q = (2 + k) % 4
            b = k % 2
            b_wait_loads(q, b)
            b_wait_store(b)
            b_compute(q, b)
            b_issue(i0 + k + 2, (q + 2) % 4, b)
        return _
    lax.fori_loop(0, (NB - 5) // 4, b_quad, None)

    b_wait_loads(2, 0)
    b_wait_store(0)
    b_compute(2, 0)
    b_issue(NB - 1, 0, 0)
    b_wait_loads(3, 1)
    b_wait_store(1)
    b_compute(3, 1)
    b_wait_loads(0, 0)
    b_wait_store(0)
    b_compute(0, 0)
    b_wait_store(0)
    b_wait_store(1)
    plsc.subcore_barrier()

    def flush_den(k, _):
        rows = pl.ds(row0 + k * ZR, ZR)
        pltpu.sync_copy(acc.at[rows], den_hbm.at[c, rows])
        return _
    lax.fori_loop(0, nchunk, flush_den, None)


def _run_edges(src2, dst2, hp2, adst2):
    mesh = plsc.VectorSubcoreMesh(core_axis_name="c", subcore_axis_name="s")
    f = functools.partial(
        pl.kernel,
        out_type=[
            jax.ShapeDtypeStruct((2, N, 128), jnp.float32),
            jax.ShapeDtypeStruct((2, N, 128), jnp.float32),
            jax.ShapeDtypeStruct((2 * E, 16), jnp.float32),
        ],
        mesh=mesh,
        scratch_types=[
            pltpu.VMEM((B,), jnp.int32),
            pltpu.VMEM((B,), jnp.int32),
            pltpu.VMEM((B,), jnp.int32),
            pltpu.VMEM((B,), jnp.int32),
            pltpu.VMEM((B,), jnp.int32),
            pltpu.VMEM((B,), jnp.int32),
            pltpu.VMEM((B,), jnp.int32),
            pltpu.VMEM((B,), jnp.int32),
            pltpu.VMEM((B, 2 * D), jnp.float32),
            pltpu.VMEM((B, 2 * D), jnp.float32),
            pltpu.VMEM((B, D), jnp.float32),
            pltpu.VMEM((B, D), jnp.float32),
            pltpu.VMEM((B, D), jnp.float32),
            pltpu.VMEM((B, D), jnp.float32),
            pltpu.VMEM((B, 16), jnp.float32),
            pltpu.VMEM((B, 16), jnp.float32),
            pltpu.VMEM_SHARED((N, D), jnp.float32),
            pltpu.SemaphoreType.DMA,
            pltpu.SemaphoreType.DMA,
            pltpu.SemaphoreType.DMA,
            pltpu.SemaphoreType.DMA,
            pltpu.SemaphoreType.DMA,
            pltpu.SemaphoreType.DMA,
            pltpu.SemaphoreType.DMA,
            pltpu.SemaphoreType.DMA,
            pltpu.SemaphoreType.DMA,
            pltpu.SemaphoreType.DMA,
        ],
    )(_edge_body)
    return f(src2, dst2, hp2, adst2)


def _epi_a_body(num_ref, den_ref, r16_ref, klin_ref, klinb_ref,
                o_ref, p_ref):
    for r in range(2):
        den128 = jnp.dot(den_ref[r][:, :16], r16_ref[...],
                         preferred_element_type=jnp.float32)
        o = jnp.maximum(num_ref[r] / (den128 + 1e-16), 0.0)
        o_ref[r] = o
        kmat = jnp.tanh(jnp.dot(o, klin_ref[...],
                                preferred_element_type=jnp.float32)
                        + klinb_ref[...])
        p_ref[0, r] = jnp.sum(kmat, axis=0)


def _run_epi_a(num, den, r16, klin, klinb):
    return pl.pallas_call(
        _epi_a_body,
        grid=(NBLK,),
        in_specs=[
            pl.BlockSpec((2, BN, 128), lambda i: (0, i, 0)),
            pl.BlockSpec((2, BN, 128), lambda i: (0, i, 0)),
            pl.BlockSpec((16, 128), lambda i: (0, 0)),
            pl.BlockSpec((D, D), lambda i: (0, 0)),
            pl.BlockSpec((1, D), lambda i: (0, 0)),
        ],
        out_specs=[
            pl.BlockSpec((2, BN, 128), lambda i: (0, i, 0)),
            pl.BlockSpec((1, 2, 128), lambda i: (i, 0, 0)),
        ],
        out_shape=[
            jax.ShapeDtypeStruct((2, N, 128), jnp.float32),
            jax.ShapeDtypeStruct((NBLK, 2, 128), jnp.float32),
        ],
    )(num, den, r16, klin, klinb)


def _epi_b_body(p_ref, q_ref, o_ref, wp_ref, bp_ref, out_ref, attn_s):
    i = pl.program_id(0)

    @pl.when(i == 0)
    def _():
        psum = jnp.sum(p_ref[...], axis=0)
        s2 = jnp.sum(psum * q_ref[...], axis=1, keepdims=True) / N
        m = jnp.max(s2, axis=0, keepdims=True)
        e = jnp.exp(s2 - m)
        attn = e / jnp.sum(e, axis=0, keepdims=True)
        attn_s[...] = jnp.broadcast_to(attn, (2, 128))

    osub = o_ref[0] * attn_s[0:1, :] + o_ref[1] * attn_s[1:2, :]
    out_ref[...] = jnp.dot(osub, wp_ref[...],
                           preferred_element_type=jnp.float32) + bp_ref[...]


def _run_epi_b(p, q2, o, wp, bp):
    return pl.pallas_call(
        _epi_b_body,
        grid=(NBLK,),
        in_specs=[
            pl.BlockSpec((NBLK, 2, 128), lambda i: (0, 0, 0)),
            pl.BlockSpec((1, 128), lambda i: (0, 0)),
            pl.BlockSpec((2, BN, 128), lambda i: (0, i, 0)),
            pl.BlockSpec((D, D), lambda i: (0, 0)),
            pl.BlockSpec((1, D), lambda i: (0, 0)),
        ],
        out_specs=pl.BlockSpec((BN, 128), lambda i: (i, 0)),
        out_shape=jax.ShapeDtypeStruct((N, 128), jnp.float32),
        scratch_shapes=[pltpu.VMEM((2, 128), jnp.float32)],
    )(p, q2, o, wp, bp)


def _att_mat(att):
    eye = jnp.eye(NH, 16, dtype=att.dtype)
    return (att[:, :, None] * eye[:, None, :]).reshape(D, 16)


def kernel(x_subject, x_electrode, edge_e2s, edge_s2s, W_proj_s, b_proj_s,
           W_proj_e, b_proj_e, att_src_e2s, att_dst_e2s, att_src_s2s,
           att_dst_s2s, k_lin_w, k_lin_b, q_sem, W_out, b_out):
    x_stack = jnp.stack([x_electrode, x_subject])
    w_stack = jnp.stack([W_proj_e, W_proj_s])
    b_stack = jnp.stack([b_proj_e, b_proj_s]).reshape(2, 1, D)
    msrc_stack = jnp.stack([_att_mat(att_src_e2s), _att_mat(att_src_s2s)])
    mdst_stack = jnp.stack([_att_mat(att_dst_e2s), _att_mat(att_dst_s2s)])

    hp, adst = _run_proj(x_stack, w_stack, b_stack, msrc_stack, mdst_stack)

    src2 = jnp.concatenate([edge_e2s[0], edge_s2s[0] + N])
    dst2 = jnp.concatenate([edge_e2s[1], edge_s2s[1]])
    num, den, _unused_ex = _run_edges(src2, dst2,
                                      hp.reshape(2 * N, 256),
                                      adst.reshape(2 * N, 128))

    r16 = jnp.repeat(jnp.eye(NH, dtype=jnp.float32), DH, axis=1)
    r16 = jnp.concatenate([r16, jnp.zeros((8, 128), jnp.float32)], axis=0)
    o, p = _run_epi_a(num, den, r16, k_lin_w, k_lin_b.reshape(1, D))

    wp = jnp.zeros((D, D), jnp.float32).at[:, :2].set(W_out)
    bp = jnp.zeros((1, D), jnp.float32).at[0, :2].set(b_out)
    out_pad = _run_epi_b(p, q_sem.reshape(1, D), o, wp, bp)
    return out_pad[:, :2]
